# baseline (speedup 1.0000x reference)
import functools

import jax
import jax.numpy as jnp
from jax import lax
from jax.experimental import pallas as pl
from jax.experimental.pallas import tpu as pltpu
from jax.experimental.pallas import tpu_sc as plsc

_NBUF = 8
_AHEAD = 6


def _make_sc_kernel(R, V, D, NS, NW, HB):
    tok_per_w = R // NW
    nhb = tok_per_w // HB
    per_blk = NS // HB
    n_bodies = nhb // _NBUF
    assert nhb % _NBUF == 0 and n_bodies >= 3
    assert _NBUF % (2 * per_blk) == 0 and D % 16 == 0 and HB <= 128

    mesh = plsc.VectorSubcoreMesh(core_axis_name="c", subcore_axis_name="s")

    @functools.partial(
        pl.kernel,
        out_type=jax.ShapeDtypeStruct((R, D), jnp.float32),
        mesh=mesh,
        scratch_types=[
            pltpu.VMEM((nhb, HB), jnp.int32),
            pltpu.VMEM((_NBUF, HB, D), jnp.float32),
            pltpu.VMEM((2, NS, D), jnp.float32),
            pltpu.VMEM((2, D), jnp.float32),
        ] + [pltpu.SemaphoreType.DMA] * (2 * _NBUF),
    )
    def sc_kernel(nt_hbm, wte_hbm, wpe_hbm, wne_hbm, out_hbm,
                  idx_v, ring, add_v, wne_v, *sems):
        wid = lax.axis_index("s") * 2 + lax.axis_index("c")
        gsem, ssem = sems[:_NBUF], sems[_NBUF:]

        def start_gather(j, q):
            pltpu.make_async_copy(
                wte_hbm.at[idx_v.at[j]], ring.at[q], gsem[q]).start()

        def wait_gather(j, q):
            pltpu.make_async_copy(
                wte_hbm.at[idx_v.at[j]], ring.at[q], gsem[q]).wait()

        def out_slice(j):
            return out_hbm.at[pl.ds((wid * nhb + j) * HB, HB)]

        def start_scatter(j, q):
            pltpu.make_async_copy(ring.at[q], out_slice(j), ssem[q]).start()

        def wait_scatter(j, q):
            pltpu.make_async_copy(ring.at[q], out_slice(j), ssem[q]).wait()

        def add_half_block(hmod):
            q = hmod % _NBUF
            p = (hmod // per_blk) % 2
            off = (hmod % per_blk) * HB

            def add_body(r, _):
                for c in range(D // 16):
                    sl = pl.ds(c * 16, 16)
                    plsc.addupdate(ring.at[q, r, sl], add_v[p, off + r, sl])
                return 0

            lax.fori_loop(0, HB, add_body, 0, unroll=4)

        pltpu.sync_copy(nt_hbm.at[wid], idx_v)
        for q in range(_AHEAD):
            start_gather(q, q)
        pltpu.sync_copy(wne_hbm, wne_v)
        pltpu.sync_copy(wpe_hbm, add_v.at[0])
        pltpu.sync_copy(wpe_hbm, add_v.at[1])

        def build_body(r, _):
            for k in range(2):
                for c in range(D // 16):
                    sl = pl.ds(c * 16, 16)
                    add_v[k, r, sl] = add_v[k, r, sl] + wne_v[k, sl]
            return 0

        lax.fori_loop(0, NS, build_body, 0, unroll=2)

        for q in range(_NBUF):
            wait_gather(q, q)
            add_half_block(q)
            start_scatter(q, q)
            if q + _AHEAD < _NBUF:
                start_gather(q + _AHEAD, (q + _AHEAD) % _NBUF)
            else:
                wait_scatter(q + _AHEAD - _NBUF, (q + _AHEAD) % _NBUF)
                start_gather(q + _AHEAD, (q + _AHEAD) % _NBUF)

        def body(i, _):
            for q in range(_NBUF):
                j = i * _NBUF + q
                wait_gather(j, q)
                add_half_block(q)
                start_scatter(j, q)
                wait_scatter(j + _AHEAD - _NBUF, (q + _AHEAD) % _NBUF)
                start_gather(j + _AHEAD, (q + _AHEAD) % _NBUF)
            return 0

        lax.fori_loop(1, n_bodies - 1, body, 0)

        for q in range(_NBUF):
            j = (n_bodies - 1) * _NBUF + q
            wait_gather(j, q)
            add_half_block(q)
            start_scatter(j, q)
            if q + _AHEAD < _NBUF:
                wait_scatter(j + _AHEAD - _NBUF, (q + _AHEAD) % _NBUF)
                start_gather(j + _AHEAD, (q + _AHEAD) % _NBUF)
        for q in range(_NBUF):
            wait_scatter((n_bodies - 1) * _NBUF + q, q)

    return sc_kernel


def kernel(neighbor_tokens, wte, wpe, wne):
    Bv, n_chunks, k_neighbors, ns = neighbor_tokens.shape
    V, D = wte.shape
    R = Bv * n_chunks * k_neighbors * ns
    NW = 32
    HB = 64
    assert R % (NW * ns) == 0 and k_neighbors == 2 and ns % HB == 0

    nt = neighbor_tokens.reshape(NW, R // (NW * HB), HB).astype(jnp.int32)
    sc = _make_sc_kernel(R, V, D, ns, NW, HB)
    out = sc(nt, wte, wpe, wne)
    return out.reshape(Bv, n_chunks, k_neighbors, ns, D)


# speedup vs baseline: 1.0687x; 1.0687x over previous
import functools

import jax
import jax.numpy as jnp
from jax import lax
from jax.experimental import pallas as pl
from jax.experimental.pallas import tpu as pltpu
from jax.experimental.pallas import tpu_sc as plsc

_NBUF = 8
_AHEAD = 6


def _make_sc_kernel(R, V, D, NS, NW, HB):
    tok_per_w = R // NW
    nhb = tok_per_w // HB
    per_blk = NS // HB
    n_bodies = nhb // _NBUF
    assert nhb % _NBUF == 0 and n_bodies >= 3
    assert _NBUF % (2 * per_blk) == 0 and D % 16 == 0 and HB <= 128

    mesh = plsc.VectorSubcoreMesh(core_axis_name="c", subcore_axis_name="s")

    @functools.partial(
        pl.kernel,
        out_type=jax.ShapeDtypeStruct((R, D), jnp.float32),
        mesh=mesh,
        scratch_types=[
            pltpu.VMEM((nhb, HB), jnp.int32),
            pltpu.VMEM((_NBUF, HB, D), jnp.float32),
            pltpu.VMEM((2, NS, D), jnp.float32),
            pltpu.VMEM((2, D), jnp.float32),
        ] + [pltpu.SemaphoreType.DMA] * (2 * _NBUF),
    )
    def sc_kernel(nt_hbm, wte_hbm, wpe_hbm, wne_hbm, out_hbm,
                  idx_v, ring, add_v, wne_v, *sems):
        wid = lax.axis_index("s") * 2 + lax.axis_index("c")
        gsem, ssem = sems[:_NBUF], sems[_NBUF:]

        def start_gather(j, q):
            pltpu.make_async_copy(
                wte_hbm.at[idx_v.at[j]], ring.at[q], gsem[q]).start()

        def wait_gather(j, q):
            pltpu.make_async_copy(
                wte_hbm.at[idx_v.at[j]], ring.at[q], gsem[q]).wait()

        def out_slice(j):
            return out_hbm.at[pl.ds((wid * nhb + j) * HB, HB)]

        def start_scatter(j, q):
            pltpu.make_async_copy(ring.at[q], out_slice(j), ssem[q]).start()

        def wait_scatter(j, q):
            pltpu.make_async_copy(ring.at[q], out_slice(j), ssem[q]).wait()

        def add_half_block(hmod):
            q = hmod % _NBUF
            p = (hmod // per_blk) % 2
            off = (hmod % per_blk) * HB

            def add_body(r, _):
                for c in range(D // 16):
                    sl = pl.ds(c * 16, 16)
                    plsc.addupdate(ring.at[q, r, sl], add_v[p, off + r, sl])
                return 0

            lax.fori_loop(0, HB, add_body, 0)

        pltpu.sync_copy(nt_hbm.at[wid], idx_v)
        for q in range(_AHEAD):
            start_gather(q, q)
        pltpu.sync_copy(wne_hbm, wne_v)
        pltpu.sync_copy(wpe_hbm, add_v.at[0])
        pltpu.sync_copy(wpe_hbm, add_v.at[1])

        def build_body(r, _):
            for k in range(2):
                for c in range(D // 16):
                    sl = pl.ds(c * 16, 16)
                    add_v[k, r, sl] = add_v[k, r, sl] + wne_v[k, sl]
            return 0

        lax.fori_loop(0, NS, build_body, 0, unroll=2)

        for q in range(_NBUF):
            wait_gather(q, q)
            add_half_block(q)
            start_scatter(q, q)
            if q + _AHEAD < _NBUF:
                start_gather(q + _AHEAD, (q + _AHEAD) % _NBUF)
            else:
                wait_scatter(q + _AHEAD - _NBUF, (q + _AHEAD) % _NBUF)
                start_gather(q + _AHEAD, (q + _AHEAD) % _NBUF)

        def body(i, _):
            for q in range(_NBUF):
                j = i * _NBUF + q
                wait_gather(j, q)
                add_half_block(q)
                start_scatter(j, q)
                wait_scatter(j + _AHEAD - _NBUF, (q + _AHEAD) % _NBUF)
                start_gather(j + _AHEAD, (q + _AHEAD) % _NBUF)
            return 0

        lax.fori_loop(1, n_bodies - 1, body, 0)

        for q in range(_NBUF):
            j = (n_bodies - 1) * _NBUF + q
            wait_gather(j, q)
            add_half_block(q)
            start_scatter(j, q)
            if q + _AHEAD < _NBUF:
                wait_scatter(j + _AHEAD - _NBUF, (q + _AHEAD) % _NBUF)
                start_gather(j + _AHEAD, (q + _AHEAD) % _NBUF)
        for q in range(_NBUF):
            wait_scatter((n_bodies - 1) * _NBUF + q, q)

    return sc_kernel


def kernel(neighbor_tokens, wte, wpe, wne):
    Bv, n_chunks, k_neighbors, ns = neighbor_tokens.shape
    V, D = wte.shape
    R = Bv * n_chunks * k_neighbors * ns
    NW = 32
    HB = 64
    assert R % (NW * ns) == 0 and k_neighbors == 2 and ns % HB == 0

    nt = neighbor_tokens.reshape(NW, R // (NW * HB), HB).astype(jnp.int32)
    sc = _make_sc_kernel(R, V, D, ns, NW, HB)
    out = sc(nt, wte, wpe, wne)
    return out.reshape(Bv, n_chunks, k_neighbors, ns, D)


# speedup vs baseline: 1.0748x; 1.0057x over previous
import functools

import jax
import jax.numpy as jnp
from jax import lax
from jax.experimental import pallas as pl
from jax.experimental.pallas import tpu as pltpu
from jax.experimental.pallas import tpu_sc as plsc

_NBUF = 8
_AHEAD = 5


def _make_sc_kernel(R, V, D, NS, NW, HB):
    tok_per_w = R // NW
    nhb = tok_per_w // HB
    per_blk = NS // HB
    n_bodies = nhb // _NBUF
    assert nhb % _NBUF == 0 and n_bodies >= 3
    assert _NBUF % (2 * per_blk) == 0 and D % 16 == 0 and HB <= 128

    mesh = plsc.VectorSubcoreMesh(core_axis_name="c", subcore_axis_name="s")

    @functools.partial(
        pl.kernel,
        out_type=jax.ShapeDtypeStruct((R, D), jnp.float32),
        mesh=mesh,
        scratch_types=[
            pltpu.VMEM((nhb, HB), jnp.int32),
            pltpu.VMEM((_NBUF, HB, D), jnp.float32),
            pltpu.VMEM((2, NS, D), jnp.float32),
            pltpu.VMEM((2, D), jnp.float32),
        ] + [pltpu.SemaphoreType.DMA] * (2 * _NBUF),
    )
    def sc_kernel(nt_hbm, wte_hbm, wpe_hbm, wne_hbm, out_hbm,
                  idx_v, ring, add_v, wne_v, *sems):
        wid = lax.axis_index("s") * 2 + lax.axis_index("c")
        gsem, ssem = sems[:_NBUF], sems[_NBUF:]

        def start_gather(j, q):
            pltpu.make_async_copy(
                wte_hbm.at[idx_v.at[j]], ring.at[q], gsem[q]).start()

        def wait_gather(j, q):
            pltpu.make_async_copy(
                wte_hbm.at[idx_v.at[j]], ring.at[q], gsem[q]).wait()

        def out_slice(j):
            return out_hbm.at[pl.ds((wid * nhb + j) * HB, HB)]

        def start_scatter(j, q):
            pltpu.make_async_copy(ring.at[q], out_slice(j), ssem[q]).start()

        def wait_scatter(j, q):
            pltpu.make_async_copy(ring.at[q], out_slice(j), ssem[q]).wait()

        def add_half_block(hmod):
            q = hmod % _NBUF
            p = (hmod // per_blk) % 2
            off = (hmod % per_blk) * HB

            def add_body(r, _):
                for c in range(D // 16):
                    sl = pl.ds(c * 16, 16)
                    plsc.addupdate(ring.at[q, r, sl], add_v[p, off + r, sl])
                return 0

            lax.fori_loop(0, HB, add_body, 0)

        pltpu.sync_copy(nt_hbm.at[wid], idx_v)
        for q in range(_AHEAD):
            start_gather(q, q)
        pltpu.sync_copy(wne_hbm, wne_v)
        pltpu.sync_copy(wpe_hbm, add_v.at[0])
        pltpu.sync_copy(wpe_hbm, add_v.at[1])

        def build_body(r, _):
            for k in range(2):
                for c in range(D // 16):
                    sl = pl.ds(c * 16, 16)
                    add_v[k, r, sl] = add_v[k, r, sl] + wne_v[k, sl]
            return 0

        lax.fori_loop(0, NS, build_body, 0, unroll=2)

        for q in range(_NBUF):
            wait_gather(q, q)
            add_half_block(q)
            start_scatter(q, q)
            if q + _AHEAD < _NBUF:
                start_gather(q + _AHEAD, (q + _AHEAD) % _NBUF)
            else:
                wait_scatter(q + _AHEAD - _NBUF, (q + _AHEAD) % _NBUF)
                start_gather(q + _AHEAD, (q + _AHEAD) % _NBUF)

        def body(i, _):
            for q in range(_NBUF):
                j = i * _NBUF + q
                wait_gather(j, q)
                add_half_block(q)
                start_scatter(j, q)
                wait_scatter(j + _AHEAD - _NBUF, (q + _AHEAD) % _NBUF)
                start_gather(j + _AHEAD, (q + _AHEAD) % _NBUF)
            return 0

        lax.fori_loop(1, n_bodies - 1, body, 0)

        for q in range(_NBUF):
            j = (n_bodies - 1) * _NBUF + q
            wait_gather(j, q)
            add_half_block(q)
            start_scatter(j, q)
            if q + _AHEAD < _NBUF:
                wait_scatter(j + _AHEAD - _NBUF, (q + _AHEAD) % _NBUF)
                start_gather(j + _AHEAD, (q + _AHEAD) % _NBUF)
        for q in range(_NBUF):
            wait_scatter((n_bodies - 1) * _NBUF + q, q)

    return sc_kernel


def kernel(neighbor_tokens, wte, wpe, wne):
    Bv, n_chunks, k_neighbors, ns = neighbor_tokens.shape
    V, D = wte.shape
    R = Bv * n_chunks * k_neighbors * ns
    NW = 32
    HB = 64
    assert R % (NW * ns) == 0 and k_neighbors == 2 and ns % HB == 0

    nt = neighbor_tokens.reshape(NW, R // (NW * HB), HB).astype(jnp.int32)
    sc = _make_sc_kernel(R, V, D, ns, NW, HB)
    out = sc(nt, wte, wpe, wne)
    return out.reshape(Bv, n_chunks, k_neighbors, ns, D)
